# baseline (speedup 1.0000x reference)
import functools

import jax
import jax.numpy as jnp
from jax import lax
from jax.experimental import pallas as pl
from jax.experimental.pallas import tpu as pltpu
from jax.experimental.pallas import tpu_sc as plsc

N_NODES = 135168
N_GRAPHS = 4096

_NC, _NS, _L = 2, 16, 16
_NW = _NC * _NS


def _sc_mesh():
    return plsc.VectorSubcoreMesh(core_axis_name="c", subcore_axis_name="s",
                                  num_cores=_NC, num_subcores=_NS)


def _zero_fill(ref, n):
    def body(i, _):
        ref[pl.ds(i * 16, 16)] = jnp.zeros((16,), jnp.float32)
        return 0
    lax.fori_loop(0, n // 16, body, 0)


@functools.lru_cache(maxsize=None)
def _make_histogram(n_idx, n_bins):
    per_w = n_idx // _NW
    assert n_idx == per_w * _NW and per_w % 128 == 0
    nb = per_w // 128
    stripe = n_bins // _NS
    assert n_bins == stripe * _NS and stripe % 16 == 0

    @functools.partial(
        pl.kernel,
        out_type=jax.ShapeDtypeStruct((_NC, n_bins), jnp.float32),
        mesh=_sc_mesh(),
        scratch_types=[
            pltpu.VMEM((nb, 128), jnp.int32),
            pltpu.VMEM((128,), jnp.float32),
            pltpu.VMEM((stripe,), jnp.float32),
            pltpu.VMEM_SHARED((n_bins,), jnp.float32),
        ])
    def hist(idx_hbm, out_hbm, idx_v, ones_v, zero_v, acc_sh):
        c = lax.axis_index("c")
        s = lax.axis_index("s")
        w = s * _NC + c

        def init_ones(i, _):
            ones_v[pl.ds(i * 16, 16)] = jnp.ones((16,), jnp.float32)
            return 0
        lax.fori_loop(0, 128 // 16, init_ones, 0)
        _zero_fill(zero_v, stripe)
        pltpu.sync_copy(zero_v, acc_sh.at[pl.ds(s * stripe, stripe)])
        pltpu.sync_copy(idx_hbm.at[pl.ds(w * nb, nb), :], idx_v)
        plsc.subcore_barrier()

        def batch(j, _):
            pltpu.sync_copy(ones_v, acc_sh.at[idx_v.at[j]], add=True)
            return 0
        lax.fori_loop(0, nb, batch, 0)
        plsc.subcore_barrier()
        pltpu.sync_copy(acc_sh.at[pl.ds(s * stripe, stripe)],
                        out_hbm.at[c, pl.ds(s * stripe, stripe)])

    return hist


@functools.lru_cache(maxsize=None)
def _make_propagate(n_edges, n_nodes, n_slabs):
    half = n_nodes // 2
    rs = half // _NS
    ys_rows = n_nodes // _NS
    ec = n_edges // _NS
    K = 32
    nbatch = ec // 128
    ngroups = nbatch // K
    assert n_edges == ec * _NS and nbatch * 128 == ec and ngroups * K == nbatch
    DUM = 2048
    zr = 528
    assert half == rs * _NS and rs % zr == 0

    @functools.partial(
        pl.kernel,
        out_type=[jax.ShapeDtypeStruct((n_nodes, 8), jnp.float32)
                  for _ in range(n_slabs)],
        mesh=_sc_mesh(),
        compiler_params=pltpu.CompilerParams(use_tc_tiling_on_sc=False),
        scratch_types=[
            pltpu.VMEM((ec,), jnp.int32),
            pltpu.VMEM((ec,), jnp.int32),
            pltpu.VMEM((K, 128), jnp.int32),
            pltpu.VMEM((K, 128), jnp.int32),
            pltpu.VMEM((K, 128, 8), jnp.float32),
            pltpu.VMEM((zr, 8), jnp.float32),
            pltpu.VMEM_SHARED((half + DUM, 8), jnp.float32),
            pltpu.SemaphoreType.DMA,
        ])
    def prop(*refs):
        zeros_hbm, src_hbm, dst_hbm = refs[0], refs[1], refs[2]
        ys = refs[3:3 + n_slabs]
        outs = refs[3 + n_slabs:3 + 2 * n_slabs]
        (srcv, dstv, sbuf, dbuf, rows, zrow, acc_sh,
         sem) = refs[3 + 2 * n_slabs:]

        c = lax.axis_index("c")
        s = lax.axis_index("s")
        lo = c * half

        pltpu.sync_copy(zeros_hbm, zrow)
        pltpu.sync_copy(src_hbm.at[pl.ds(s * ec, ec)], srcv)
        pltpu.sync_copy(dst_hbm.at[pl.ds(s * ec, ec)], dstv)

        for slab in range(n_slabs):
            for q in range(rs // zr):
                pltpu.sync_copy(
                    zrow, acc_sh.at[pl.ds(s * rs + q * zr, zr), :])
            plsc.subcore_barrier()

            for g in range(ngroups):
                def fire(p, _):
                    j = g * K + p
                    for k in range(8):
                        e0 = j * 128 + k * 16
                        d16 = dstv[pl.ds(e0, 16)]
                        m = (d16 >= lo) & (d16 < lo + half)
                        dum = half + ((e0 + lax.iota(jnp.int32, 16))
                                      & (DUM - 1))
                        sbuf[p, pl.ds(k * 16, 16)] = srcv[pl.ds(e0, 16)]
                        dbuf[p, pl.ds(k * 16, 16)] = jnp.where(
                            m, d16 - lo, dum)
                    pltpu.make_async_copy(
                        ys[slab].at[sbuf.at[p]], rows.at[p], sem).start()
                    return 0
                lax.fori_loop(0, K, fire, 0)

                def drain_fire_s(p, _):
                    pltpu.make_async_copy(
                        ys[slab].at[sbuf.at[p]], rows.at[p], sem).wait()
                    pltpu.sync_copy(rows.at[p], acc_sh.at[dbuf.at[p]],
                                    add=True)
                    return 0
                lax.fori_loop(0, K, drain_fire_s, 0)

            plsc.subcore_barrier()
            pltpu.sync_copy(acc_sh.at[pl.ds(s * rs, rs), :],
                            outs[slab].at[pl.ds(lo + s * rs, rs), :])
            plsc.subcore_barrier()

    return prop


def _head_body(pooled_ref, cnt_ref, a_ref, wp_ref, bp_ref, wc1_ref, bc1_ref,
               wc2_ref, bc2_ref, out_ref):
    sums = pooled_ref[...]
    cnt = cnt_ref[...]
    pooled = sums / jnp.maximum(cnt, 1.0)
    pooled = pooled @ wp_ref[...] + bp_ref[...]
    h = jnp.concatenate([pooled, a_ref[...]], axis=1)
    h = jnp.maximum(h @ wc1_ref[...] + bc1_ref[...], 0.0)
    out_ref[...] = h @ wc2_ref[...] + bc2_ref[...]


def _head(sums, cnt, a, Wp, bp, Wc1, bc1, Wc2, bc2):
    B = sums.shape[0]
    return pl.pallas_call(
        _head_body,
        out_shape=jax.ShapeDtypeStruct((B, Wc2.shape[1]), jnp.float32),
    )(sums, cnt[:, None], a, Wp, bp[None, :], Wc1, bc1[None, :], Wc2,
      bc2[None, :])


def kernel(x, edge_index, batch, angles, W1, b1, W2, b2, Wp, bp, Wa1, ba1,
           Wa2, ba2, Wc1, bc1, Wc2, bc2):
    src, dst = edge_index[0], edge_index[1]

    indeg2 = _make_histogram(dst.shape[0], N_NODES)(dst.reshape(-1, 128))
    deg = indeg2[0] + indeg2[1] + 1.0
    dinv = lax.rsqrt(deg)

    E = src.shape[0]
    prop1 = _make_propagate(E, N_NODES, 1)
    prop8 = _make_propagate(E, N_NODES, 8)
    zeros8 = jnp.zeros((528, 8), jnp.float32)

    y1 = jnp.concatenate(
        [x * dinv[:, None], jnp.zeros((N_NODES, 4), jnp.float32)], axis=1)
    s1 = prop1(zeros8, src, dst, y1)[0]
    px = dinv[:, None] * s1[:, :4] + (dinv * dinv)[:, None] * x
    h1 = jax.nn.relu(px @ W1 + b1)

    z = h1 @ W2
    y2 = z * dinv[:, None]
    s2 = prop8(zeros8, src, dst,
               *[y2[:, 8 * i:8 * i + 8] for i in range(8)])
    s2 = jnp.concatenate(s2, axis=1)
    h2 = jax.nn.relu(dinv[:, None] * s2 + (dinv * dinv)[:, None] * z + b2)

    sums = jax.ops.segment_sum(h2, batch, num_segments=N_GRAPHS)
    cnt = jax.ops.segment_sum(jnp.ones((N_NODES,), jnp.float32), batch,
                              num_segments=N_GRAPHS)

    a = jax.nn.relu(angles @ Wa1 + ba1)
    a = jax.nn.relu(a @ Wa2 + ba2)

    return _head(sums, cnt, a, Wp, bp, Wc1, bc1, Wc2, bc2)


# speedup vs baseline: 1.1833x; 1.1833x over previous
import functools

import jax
import jax.numpy as jnp
from jax import lax
from jax.experimental import pallas as pl
from jax.experimental.pallas import tpu as pltpu
from jax.experimental.pallas import tpu_sc as plsc

N_NODES = 135168
N_GRAPHS = 4096

_NC, _NS, _L = 2, 16, 16
_NW = _NC * _NS


def _sc_mesh():
    return plsc.VectorSubcoreMesh(core_axis_name="c", subcore_axis_name="s",
                                  num_cores=_NC, num_subcores=_NS)


def _zero_fill(ref, n):
    def body(i, _):
        ref[pl.ds(i * 16, 16)] = jnp.zeros((16,), jnp.float32)
        return 0
    lax.fori_loop(0, n // 16, body, 0)


@functools.lru_cache(maxsize=None)
def _make_histogram(n_idx, n_bins):
    per_w = n_idx // _NW
    assert n_idx == per_w * _NW and per_w % 128 == 0
    nb = per_w // 128
    stripe = n_bins // _NS
    assert n_bins == stripe * _NS and stripe % 16 == 0

    @functools.partial(
        pl.kernel,
        out_type=jax.ShapeDtypeStruct((_NC, n_bins), jnp.float32),
        mesh=_sc_mesh(),
        scratch_types=[
            pltpu.VMEM((nb, 128), jnp.int32),
            pltpu.VMEM((128,), jnp.float32),
            pltpu.VMEM((stripe,), jnp.float32),
            pltpu.VMEM_SHARED((n_bins,), jnp.float32),
        ])
    def hist(idx_hbm, out_hbm, idx_v, ones_v, zero_v, acc_sh):
        c = lax.axis_index("c")
        s = lax.axis_index("s")
        w = s * _NC + c

        def init_ones(i, _):
            ones_v[pl.ds(i * 16, 16)] = jnp.ones((16,), jnp.float32)
            return 0
        lax.fori_loop(0, 128 // 16, init_ones, 0)
        _zero_fill(zero_v, stripe)
        pltpu.sync_copy(zero_v, acc_sh.at[pl.ds(s * stripe, stripe)])
        pltpu.sync_copy(idx_hbm.at[w], idx_v)
        plsc.subcore_barrier()

        def batch(j, _):
            pltpu.sync_copy(ones_v, acc_sh.at[idx_v.at[j]], add=True)
            return 0
        lax.fori_loop(0, nb, batch, 0)
        plsc.subcore_barrier()
        pltpu.sync_copy(acc_sh.at[pl.ds(s * stripe, stripe)],
                        out_hbm.at[c, pl.ds(s * stripe, stripe)])

    return hist


@functools.lru_cache(maxsize=None)
def _make_propagate(n_edges, n_nodes, n_slabs):
    half = n_nodes // 2
    rs = half // _NS
    ys_rows = n_nodes // _NS
    ec = n_edges // _NS
    K = 32
    nbatch = ec // 128
    ngroups = nbatch // K
    assert n_edges == ec * _NS and nbatch * 128 == ec and ngroups * K == nbatch
    DUM = 2048
    zr = 528
    assert half == rs * _NS and rs % zr == 0

    @functools.partial(
        pl.kernel,
        out_type=[jax.ShapeDtypeStruct((n_nodes, 8), jnp.float32)
                  for _ in range(n_slabs)],
        mesh=_sc_mesh(),
        compiler_params=pltpu.CompilerParams(use_tc_tiling_on_sc=False),
        scratch_types=[
            pltpu.VMEM((ec,), jnp.int32),
            pltpu.VMEM((ec,), jnp.int32),
            pltpu.VMEM((K, 128), jnp.int32),
            pltpu.VMEM((K, 128), jnp.int32),
            pltpu.VMEM((K, 128, 8), jnp.float32),
            pltpu.VMEM((zr, 8), jnp.float32),
            pltpu.VMEM_SHARED((half + DUM, 8), jnp.float32),
            pltpu.SemaphoreType.DMA,
        ])
    def prop(*refs):
        zeros_hbm, src_hbm, dst_hbm = refs[0], refs[1], refs[2]
        ys = refs[3:3 + n_slabs]
        outs = refs[3 + n_slabs:3 + 2 * n_slabs]
        (srcv, dstv, sbuf, dbuf, rows, zrow, acc_sh,
         sem) = refs[3 + 2 * n_slabs:]

        c = lax.axis_index("c")
        s = lax.axis_index("s")
        lo = c * half

        pltpu.sync_copy(zeros_hbm, zrow)
        pltpu.sync_copy(src_hbm.at[pl.ds(s * ec, ec)], srcv)
        pltpu.sync_copy(dst_hbm.at[pl.ds(s * ec, ec)], dstv)

        for slab in range(n_slabs):
            for q in range(rs // zr):
                pltpu.sync_copy(
                    zrow, acc_sh.at[pl.ds(s * rs + q * zr, zr), :])
            plsc.subcore_barrier()

            for g in range(ngroups):
                def fire(p, _):
                    j = g * K + p
                    for k in range(8):
                        e0 = j * 128 + k * 16
                        d16 = dstv[pl.ds(e0, 16)]
                        m = (d16 >= lo) & (d16 < lo + half)
                        dum = half + ((e0 + lax.iota(jnp.int32, 16))
                                      & (DUM - 1))
                        sbuf[p, pl.ds(k * 16, 16)] = srcv[pl.ds(e0, 16)]
                        dbuf[p, pl.ds(k * 16, 16)] = jnp.where(
                            m, d16 - lo, dum)
                    pltpu.make_async_copy(
                        ys[slab].at[sbuf.at[p]], rows.at[p], sem).start()
                    return 0
                lax.fori_loop(0, K, fire, 0)

                def drain_fire_s(p, _):
                    pltpu.make_async_copy(
                        ys[slab].at[sbuf.at[p]], rows.at[p], sem).wait()
                    pltpu.sync_copy(rows.at[p], acc_sh.at[dbuf.at[p]],
                                    add=True)
                    return 0
                lax.fori_loop(0, K, drain_fire_s, 0)

            plsc.subcore_barrier()
            pltpu.sync_copy(acc_sh.at[pl.ds(s * rs, rs), :],
                            outs[slab].at[pl.ds(lo + s * rs, rs), :])
            plsc.subcore_barrier()

    return prop


@functools.lru_cache(maxsize=None)
def _make_pool(n_nodes, n_graphs):
    npt = n_nodes // _NW
    nb = npt // 128
    assert n_nodes == npt * _NW and nb * 128 == npt
    K = 8
    gs = n_graphs // _NS
    zr = gs

    @functools.partial(
        pl.kernel,
        out_type=jax.ShapeDtypeStruct((_NC, n_graphs, 64), jnp.float32),
        mesh=_sc_mesh(),
        compiler_params=pltpu.CompilerParams(use_tc_tiling_on_sc=False),
        scratch_types=[
            pltpu.VMEM((npt,), jnp.int32),
            pltpu.VMEM((K, 128), jnp.int32),
            pltpu.VMEM((K, 128, 64), jnp.float32),
            pltpu.VMEM((zr, 64), jnp.float32),
            pltpu.VMEM_SHARED((n_graphs, 64), jnp.float32),
            pltpu.SemaphoreType.DMA,
        ])
    def pool(zeros_hbm, h_hbm, bid_hbm, out_hbm, bidv, dbuf, rows, zrow,
             acc_sh, sem):
        c = lax.axis_index("c")
        s = lax.axis_index("s")
        w = c * _NS + s
        base = pl.multiple_of(w * npt, 128)

        pltpu.sync_copy(zeros_hbm, zrow)
        pltpu.sync_copy(bid_hbm.at[pl.ds(base, npt)], bidv)
        pltpu.sync_copy(zrow, acc_sh.at[pl.ds(s * gs, gs), :])
        plsc.subcore_barrier()

        for g0 in range(0, nb, K):
            kk = min(K, nb - g0)

            def fire(p, _):
                j = g0 + p
                for k in range(8):
                    dbuf[p, pl.ds(k * 16, 16)] = bidv[
                        pl.ds(j * 128 + k * 16, 16)]
                pltpu.make_async_copy(
                    h_hbm.at[pl.ds(pl.multiple_of(base + j * 128, 128),
                                   128), :], rows.at[p], sem).start()
                return 0
            lax.fori_loop(0, kk, fire, 0)

            def drain(p, _):
                j = g0 + p
                pltpu.make_async_copy(
                    h_hbm.at[pl.ds(pl.multiple_of(base + j * 128, 128),
                                   128), :], rows.at[p], sem).wait()
                pltpu.sync_copy(rows.at[p], acc_sh.at[dbuf.at[p]],
                                add=True)
                return 0
            lax.fori_loop(0, kk, drain, 0)

        plsc.subcore_barrier()
        so = pl.multiple_of(s * gs, gs)
        pltpu.sync_copy(acc_sh.at[pl.ds(so, gs), :],
                        out_hbm.at[c, pl.ds(so, gs), :])

    return pool


def _head_body(pooled_ref, cnt_ref, a_ref, wp_ref, bp_ref, wc1_ref, bc1_ref,
               wc2_ref, bc2_ref, out_ref):
    sums = pooled_ref[...]
    cnt = cnt_ref[...]
    pooled = sums / jnp.maximum(cnt, 1.0)
    pooled = pooled @ wp_ref[...] + bp_ref[...]
    h = jnp.concatenate([pooled, a_ref[...]], axis=1)
    h = jnp.maximum(h @ wc1_ref[...] + bc1_ref[...], 0.0)
    out_ref[...] = h @ wc2_ref[...] + bc2_ref[...]


def _head(sums, cnt, a, Wp, bp, Wc1, bc1, Wc2, bc2):
    B = sums.shape[0]
    return pl.pallas_call(
        _head_body,
        out_shape=jax.ShapeDtypeStruct((B, Wc2.shape[1]), jnp.float32),
    )(sums, cnt[:, None], a, Wp, bp[None, :], Wc1, bc1[None, :], Wc2,
      bc2[None, :])


def kernel(x, edge_index, batch, angles, W1, b1, W2, b2, Wp, bp, Wa1, ba1,
           Wa2, ba2, Wc1, bc1, Wc2, bc2):
    src, dst = edge_index[0], edge_index[1]

    indeg2 = _make_histogram(dst.shape[0], N_NODES)(
        dst.reshape(_NW, -1, 128))
    deg = indeg2[0] + indeg2[1] + 1.0
    dinv = lax.rsqrt(deg)

    E = src.shape[0]
    prop1 = _make_propagate(E, N_NODES, 1)
    prop8 = _make_propagate(E, N_NODES, 8)
    zeros8 = jnp.zeros((528, 8), jnp.float32)

    y1 = jnp.concatenate(
        [x * dinv[:, None], jnp.zeros((N_NODES, 4), jnp.float32)], axis=1)
    s1 = prop1(zeros8, src, dst, y1)[0]
    px = dinv[:, None] * s1[:, :4] + (dinv * dinv)[:, None] * x
    h1 = jax.nn.relu(px @ W1 + b1)

    z = h1 @ W2
    y2 = z * dinv[:, None]
    s2 = prop8(zeros8, src, dst,
               *[y2[:, 8 * i:8 * i + 8] for i in range(8)])
    s2 = jnp.concatenate(s2, axis=1)
    h2 = jax.nn.relu(dinv[:, None] * s2 + (dinv * dinv)[:, None] * z + b2)

    sums2 = _make_pool(N_NODES, N_GRAPHS)(
        jnp.zeros((N_GRAPHS // _NS, 64), jnp.float32), h2, batch)
    sums = sums2[0] + sums2[1]
    cnt2 = _make_histogram(N_NODES, N_GRAPHS)(
        batch.reshape(_NW, -1, 128))
    cnt = cnt2[0] + cnt2[1]

    a = jax.nn.relu(angles @ Wa1 + ba1)
    a = jax.nn.relu(a @ Wa2 + ba2)

    return _head(sums, cnt, a, Wp, bp, Wc1, bc1, Wc2, bc2)
